# baseline (speedup 1.0000x reference)
import jax
import jax.numpy as jnp
from jax.experimental import pallas as pl
from jax.experimental.pallas import tpu as pltpu
from jax.experimental.pallas import tpu_sc as plsc

B = 8
T_AUDIO = 160000
HOP = 320
D = 256
K = 1024
T = T_AUDIO // HOP
TB = 4
R = TB * T
CPAD = 2048
NPAD = (B // TB) * CPAD
W = 128


def _codes_kernel(frames_ref, w_ref, cb_ref, codes_ref, ct_ref, csq_ref):
    b = pl.program_id(0)

    @pl.when(b == 0)
    def _init():
        ct = jnp.transpose(cb_ref[...])
        ct_ref[...] = ct
        csq_ref[...] = jnp.sum(ct * ct, axis=0, keepdims=True)

    frames = frames_ref[...].reshape(R, HOP)
    z = jnp.dot(frames, w_ref[...], preferred_element_type=jnp.float32)
    dots = jnp.dot(z, ct_ref[...], preferred_element_type=jnp.float32)
    z_sq = jnp.sum(z * z, axis=1, keepdims=True)
    dist = (z_sq - 2.0 * dots) + csq_ref[...]

    codes = jnp.argmin(dist, axis=1)
    codes_row = codes.reshape(1, R).astype(jnp.int32)
    codes_ref[...] = jnp.zeros((1, 1, CPAD), jnp.int32)
    codes_ref[:, :, pl.ds(0, R)] = codes_row.reshape(1, 1, R)


def _transpose_kernel(x_ref, out_ref):
    xt = jnp.transpose(x_ref[...])
    for i in range(TB):
        out_ref[i] = xt[:, i * T:(i + 1) * T]


def _sc_gather(codebook, idx):
    mesh = plsc.VectorSubcoreMesh(core_axis_name="c", subcore_axis_name="s")

    @pl.kernel(out_type=jax.ShapeDtypeStruct((CPAD, D), jnp.float32),
               mesh=mesh)
    def gk(cb_hbm, i_hbm, o_hbm):
        def body(i_vmem, o_vmem):
            pltpu.sync_copy(cb_hbm.at[i_vmem.at[0]], o_vmem)

        pltpu.emit_pipeline(
            body,
            grid=(CPAD // W,),
            in_specs=[pl.BlockSpec((1, W), lambda i: (0, i))],
            out_specs=[pl.BlockSpec((W, D), lambda i: (i, 0))],
            core_axis_name=("c", "s"),
            dimension_semantics=(pltpu.PARALLEL,),
        )(i_hbm, o_hbm)

    return gk(codebook, idx)


def kernel(source, W_enc, codebook):
    frames = source.reshape(B, T, HOP)
    outs = []
    for c in range(B // TB):
        fr = jax.lax.slice_in_dim(frames, c * TB, (c + 1) * TB, axis=0)
        codes = pl.pallas_call(
            _codes_kernel,
            grid=(1,),
            in_specs=[
                pl.BlockSpec((TB, T, HOP), lambda b: (0, 0, 0)),
                pl.BlockSpec((HOP, D), lambda b: (0, 0)),
                pl.BlockSpec((K, D), lambda b: (0, 0)),
            ],
            out_specs=pl.BlockSpec((1, 1, CPAD), lambda b: (0, 0, 0)),
            out_shape=jax.ShapeDtypeStruct((1, 1, CPAD), jnp.int32),
            scratch_shapes=[
                pltpu.VMEM((D, K), jnp.float32),
                pltpu.VMEM((1, K), jnp.float32),
            ],
            compiler_params=pltpu.CompilerParams(
                dimension_semantics=("arbitrary",),
            ),
        )(fr, W_enc, codebook)

        gathered = _sc_gather(codebook, codes.reshape(1, CPAD))

        out_c = pl.pallas_call(
            _transpose_kernel,
            grid=(1,),
            in_specs=[pl.BlockSpec((CPAD, D), lambda i: (0, 0))],
            out_specs=pl.BlockSpec((TB, D, T), lambda i: (0, 0, 0)),
            out_shape=jax.ShapeDtypeStruct((TB, D, T), jnp.float32),
            compiler_params=pltpu.CompilerParams(
                dimension_semantics=("arbitrary",),
            ),
        )(gathered)
        outs.append(out_c)
    return jnp.concatenate(outs, axis=0)


# speedup vs baseline: 2.7293x; 2.7293x over previous
import jax
import jax.numpy as jnp
from jax.experimental import pallas as pl
from jax.experimental.pallas import tpu as pltpu

B = 8
T_AUDIO = 160000
HOP = 320
D = 256
K = 1024
T = T_AUDIO // HOP
TB = 4
R = TB * T


def _fused_kernel(frames_ref, w_ref, cb_ref, out_ref, ct_ref, csq_ref):
    b = pl.program_id(0)

    @pl.when(b == 0)
    def _init():
        ct = jnp.transpose(cb_ref[...])
        ct_ref[...] = ct.astype(jnp.bfloat16)
        csq_ref[...] = jnp.sum(ct * ct, axis=0, keepdims=True)

    frames = frames_ref[...].reshape(R, HOP)
    z = jnp.dot(frames, w_ref[...], preferred_element_type=jnp.float32)
    zb = z.astype(jnp.bfloat16)
    dots = jnp.dot(zb, ct_ref[...], preferred_element_type=jnp.float32)
    z_sq = jnp.sum(z * z, axis=1, keepdims=True)
    dist = (z_sq - 2.0 * dots) + csq_ref[...]

    codes = jnp.argmin(dist, axis=1)
    codes_row = codes.reshape(1, R).astype(jnp.int32)

    oh = (jax.lax.broadcasted_iota(jnp.int32, (K, R), 0) == codes_row)
    oh = oh.astype(jnp.bfloat16)
    dec = jax.lax.dot_general(
        ct_ref[...], oh, (((1,), (0,)), ((), ())),
        preferred_element_type=jnp.float32)
    for i in range(TB):
        out_ref[i] = dec[:, i * T:(i + 1) * T]


def kernel(source, W_enc, codebook):
    frames = source.reshape(B, T, HOP)
    return pl.pallas_call(
        _fused_kernel,
        grid=(B // TB,),
        in_specs=[
            pl.BlockSpec((TB, T, HOP), lambda b: (b, 0, 0)),
            pl.BlockSpec((HOP, D), lambda b: (0, 0)),
            pl.BlockSpec((K, D), lambda b: (0, 0)),
        ],
        out_specs=pl.BlockSpec((TB, D, T), lambda b: (b, 0, 0)),
        out_shape=jax.ShapeDtypeStruct((B, D, T), jnp.float32),
        scratch_shapes=[
            pltpu.VMEM((D, K), jnp.bfloat16),
            pltpu.VMEM((1, K), jnp.float32),
        ],
        compiler_params=pltpu.CompilerParams(
            dimension_semantics=("arbitrary",),
        ),
    )(frames, W_enc, codebook)
